# Initial kernel scaffold; baseline (speedup 1.0000x reference)
import jax
import jax.numpy as jnp
from jax.experimental import pallas as pl


def kernel(vocab_ids, labels, edge_lists, emb, Wq, Wk, Wv, Wo, ln1_g, ln1_b, W1, b1, W2, b2, ln2_g, ln2_b, gate_W, gate_b, tr_W, tr_b):
    raise NotImplementedError("write your pallas kernel here")


# speedup vs baseline: 12.6643x; 12.6643x over previous
import jax
import jax.numpy as jnp
import numpy as np
from jax import lax
from jax.experimental import pallas as pl
from jax.experimental.pallas import tpu as pltpu
from jax.experimental.pallas import tpu_sc as plsc

N = 10000
E = 320000
D = 128
H = 8
DH = D // H
L = 4
C = 104
FF = 512

NSC = 2
NTILE = 16
NW = NSC * NTILE
EPAD = 327680
EPW = EPAD // NW
CHUNK = 32
NCHUNK = EPW // CHUNK
NACC = 10240
RPT = NACC // NTILE
NDEN = NACC // 8
RPTD = NDEN // NTILE

NPAD = 10240
GPW = NPAD // NW


def _emb_body(ids_hbm, emb_hbm, out_hbm, idx_v, rows_v, sem):
    c = lax.axis_index("c")
    s = lax.axis_index("s")
    w = c * NTILE + s
    base = pl.multiple_of(w * GPW, 8)
    pltpu.sync_copy(ids_hbm.at[pl.ds(base, GPW)], idx_v)
    pltpu.async_copy(emb_hbm.at[idx_v], rows_v, sem).wait()
    pltpu.sync_copy(rows_v, out_hbm.at[pl.ds(base, GPW)])


def _emb_gather(ids_pad, emb):
    mesh = plsc.VectorSubcoreMesh(core_axis_name="c", subcore_axis_name="s")
    return pl.kernel(
        _emb_body,
        out_type=jax.ShapeDtypeStruct((NPAD, D), jnp.float32),
        mesh=mesh,
        scratch_types=[
            pltpu.VMEM((GPW,), jnp.int32),
            pltpu.VMEM((GPW, D), jnp.float32),
            pltpu.SemaphoreType.DMA,
        ],
    )(ids_pad, emb)


def _edge_body(q_hbm, kv_hbm, dstg_hbm, dsts_hbm, dst8_hbm, src_hbm,
               out_hbm, den_hbm,
               dstv, dsts, srcv, dstv8, mbuf, qd, kvs, msg, denb, denv, acc,
               accd, sem):
    c = lax.axis_index("c")
    s = lax.axis_index("s")
    w = c * NTILE + s

    def zero_row(i, carry):
        for t in range(D // 16):
            msg[i, pl.ds(t * 16, 16)] = jnp.zeros((16,), jnp.float32)
        return carry

    lax.fori_loop(0, CHUNK, zero_row, 0)

    def zero_acc(t, carry):
        off = pl.multiple_of(s * RPT + t * CHUNK, 8)
        pltpu.sync_copy(msg.at[pl.ds(0, CHUNK)], acc.at[pl.ds(off, CHUNK)])
        return carry

    lax.fori_loop(0, RPT // CHUNK, zero_acc, 0)

    def zero_accd(t, carry):
        off = pl.multiple_of(s * RPTD + t * 16, 8)
        pltpu.sync_copy(msg.at[pl.ds(0, 16)], accd.at[pl.ds(off, 16)])
        return carry

    lax.fori_loop(0, RPTD // 16, zero_accd, 0)
    plsc.subcore_barrier()

    lane = lax.iota(jnp.int32, 16)
    zidx = lane - lane
    lanef = lane.astype(jnp.float32)
    onehot = [jnp.maximum(1.0 - jnp.abs(lanef - float(hh)), 0.0)
              for hh in range(H)]
    dn = lax.GatherDimensionNumbers(offset_dims=(), collapsed_slice_dims=(0,),
                                    start_index_map=(0,))

    def _perm(v, idx):
        return lax.gather(v, idx[:, None], dn, (1,),
                          mode=lax.GatherScatterMode.PROMISE_IN_BOUNDS)

    ebase = w * EPW

    def chunk_body(j, carry):
        cb = pl.multiple_of(ebase + j * CHUNK, 8)
        pltpu.sync_copy(dstg_hbm.at[pl.ds(cb, CHUNK)], dstv)
        pltpu.sync_copy(dsts_hbm.at[pl.ds(cb, CHUNK)], dsts)
        pltpu.sync_copy(dst8_hbm.at[pl.ds(cb, CHUNK)], dstv8)
        pltpu.sync_copy(src_hbm.at[pl.ds(cb, CHUNK)], srcv)
        cp1 = pltpu.async_copy(q_hbm.at[dstv], qd, sem)
        cp2 = pltpu.async_copy(kv_hbm.at[srcv], kvs, sem)
        cp1.wait()
        cp2.wait()

        for i in range(CHUNK // 16):
            dvec = dsts[pl.ds(i * 16, 16)]
            mbuf[i, pl.ds(0, 16)] = jnp.bitwise_and(dvec, 7).astype(
                jnp.float32)

        def edge_body(e, carry2):
            den = jnp.zeros((16,), jnp.float32)
            for h in range(H):
                vq = qd[e, pl.ds(h * 16, 16)]
                vk = kvs[e, pl.ds(h * 16, 16)]
                vv = kvs[e, pl.ds(D + h * 16, 16)]
                p = vq * vk
                for k in (8, 4, 2, 1):
                    p = p + _perm(p, jnp.bitwise_xor(lane, k))
                ex = jnp.exp(p * 0.25)
                msg[e, pl.ds(h * 16, 16)] = ex * vv
                den = den + ex * onehot[h]
            dv = mbuf[lax.shift_right_logical(e, 4), pl.ds(0, 16)]
            mf = _perm(dv, jnp.full((16,), jnp.bitwise_and(e, 15),
                                    jnp.int32))
            for j in range(8):
                w = jnp.maximum(1.0 - jnp.abs(mf - float(j)), 0.0)
                denb[e, pl.ds(j * 16, 16)] = den * w
            return carry2

        lax.fori_loop(0, CHUNK, edge_body, 0)
        pltpu.sync_copy(msg, acc.at[dsts], add=True)
        pltpu.sync_copy(denb, accd.at[dstv8], add=True)
        return carry

    lax.fori_loop(0, NCHUNK, chunk_body, 0)
    plsc.subcore_barrier()
    pltpu.sync_copy(acc.at[pl.ds(s * RPT, RPT)],
                    out_hbm.at[c].at[pl.ds(s * RPT, RPT)])
    def unpack(t, carry):
        offp = pl.multiple_of(s * RPTD + t * 8, 8)
        pltpu.sync_copy(accd.at[pl.ds(offp, 8)], qd.at[pl.ds(0, 8)])
        for r in range(8):
            for m2 in range(8):
                denv[r * 8 + m2, pl.ds(0, 16)] = qd[r, pl.ds(m2 * 16, 16)]
        offn = pl.multiple_of(s * RPT + t * 64, 8)
        pltpu.sync_copy(denv, den_hbm.at[c].at[pl.ds(offn, 64)])
        return carry

    lax.fori_loop(0, RPTD // 8, unpack, 0)


def _edge_attn(q, kv, dstg, dsts_a, dst8_a, src):
    mesh = plsc.VectorSubcoreMesh(core_axis_name="c", subcore_axis_name="s")
    return pl.kernel(
        _edge_body,
        out_type=(jax.ShapeDtypeStruct((NSC, NACC, D), jnp.float32),
                  jax.ShapeDtypeStruct((NSC, NACC, D), jnp.float32)),
        mesh=mesh,
        scratch_types=[
            pltpu.VMEM((CHUNK,), jnp.int32),
            pltpu.VMEM((CHUNK,), jnp.int32),
            pltpu.VMEM((CHUNK,), jnp.int32),
            pltpu.VMEM((CHUNK,), jnp.int32),
            pltpu.VMEM((CHUNK // 16, 16), jnp.float32),
            pltpu.VMEM((CHUNK, D), jnp.float32),
            pltpu.VMEM((CHUNK, 2 * D), jnp.float32),
            pltpu.VMEM((CHUNK, D), jnp.float32),
            pltpu.VMEM((CHUNK, D), jnp.float32),
            pltpu.VMEM((64, D), jnp.float32),
            pltpu.VMEM_SHARED((NACC, D), jnp.float32),
            pltpu.VMEM_SHARED((NDEN, D), jnp.float32),
            pltpu.SemaphoreType.DMA,
        ],
    )(q, kv, dstg, dsts_a, dst8_a, src)


BN = 1000
NB = N // BN


def _qkv_body(h_ref, w_ref, q_ref, kv_ref):
    res = jnp.dot(h_ref[...], w_ref[...], preferred_element_type=jnp.float32)
    q_ref[...] = res[:, :D]
    kv_ref[...] = res[:, D:]


def _tc_qkv(h, wqkv):
    return pl.pallas_call(
        _qkv_body,
        grid=(NB,),
        in_specs=[
            pl.BlockSpec((BN, D), lambda i: (i, 0)),
            pl.BlockSpec((D, 3 * D), lambda i: (0, 0)),
        ],
        out_specs=[
            pl.BlockSpec((BN, D), lambda i: (i, 0)),
            pl.BlockSpec((BN, 2 * D), lambda i: (i, 0)),
        ],
        out_shape=[
            jax.ShapeDtypeStruct((N, D), jnp.float32),
            jax.ShapeDtypeStruct((N, 2 * D), jnp.float32),
        ],
    )(h, wqkv)


def _ln(x, g, b):
    mu = jnp.mean(x, axis=-1, keepdims=True)
    r = x - mu
    var = jnp.mean(r * r, axis=-1, keepdims=True)
    return r / jnp.sqrt(var + 1e-5) * g + b


def _post_body(h_ref, nd0_ref, nd1_ref, de0_ref, de1_ref, wo_ref,
               ln1g_ref, ln1b_ref,
               w1_ref, b1_ref, w2_ref, b2_ref, ln2g_ref, ln2b_ref,
               sden_ref, out_ref):
    num = (nd0_ref[...] + nd1_ref[...])[0]
    den16 = (de0_ref[...] + de1_ref[...])[0][:, :16]
    denb = jnp.dot(den16, sden_ref[...], preferred_element_type=jnp.float32)
    agg = num / (denb + 1e-9)
    attn = jnp.dot(agg, wo_ref[...], preferred_element_type=jnp.float32)
    x = _ln(h_ref[...] + attn, ln1g_ref[...], ln1b_ref[...])
    ffp = jnp.maximum(
        jnp.dot(x, w1_ref[...], preferred_element_type=jnp.float32)
        + b1_ref[...], 0.0)
    ff = jnp.dot(ffp, w2_ref[...], preferred_element_type=jnp.float32) \
        + b2_ref[...]
    out_ref[...] = _ln(x + ff, ln2g_ref[...], ln2b_ref[...])


def _tc_post(h, nummsg, dens, wo, ln1g, ln1b, w1, b1, w2, b2, ln2g, ln2b,
             sden):
    full = lambda r, c: pl.BlockSpec((r, c), lambda i: (0, 0))
    return pl.pallas_call(
        _post_body,
        grid=(NB,),
        in_specs=[
            pl.BlockSpec((BN, D), lambda i: (i, 0)),
            pl.BlockSpec((1, BN, D), lambda i: (0, i, 0)),
            pl.BlockSpec((1, BN, D), lambda i: (1, i, 0)),
            pl.BlockSpec((1, BN, D), lambda i: (0, i, 0)),
            pl.BlockSpec((1, BN, D), lambda i: (1, i, 0)),
            full(D, D),
            full(1, D), full(1, D),
            full(D, FF), full(1, FF),
            full(FF, D), full(1, D),
            full(1, D), full(1, D),
            full(16, D),
        ],
        out_specs=pl.BlockSpec((BN, D), lambda i: (i, 0)),
        out_shape=jax.ShapeDtypeStruct((N, D), jnp.float32),
    )(h, nummsg, nummsg, dens, dens, wo, ln1g, ln1b, w1, b1, w2, b2,
      ln2g, ln2b, sden)


def _readout_body(ri_ref, h_ref, wg1_ref, wg2_ref, gb_ref, wt_ref, tb_ref,
                  lab_ref, logit_ref, corr_ref, acc_ref):
    i = pl.program_id(0)
    ri = ri_ref[...]
    hh = h_ref[...]
    g = jax.nn.sigmoid(
        jnp.dot(ri, wg1_ref[...], preferred_element_type=jnp.float32)
        + jnp.dot(hh, wg2_ref[...], preferred_element_type=jnp.float32)
        + gb_ref[...])
    t = jnp.dot(hh, wt_ref[...], preferred_element_type=jnp.float32) \
        + tb_ref[...]
    lg = g * t
    logit_ref[...] = lg
    cols = lax.broadcasted_iota(jnp.int32, lg.shape, 1)
    lgm = jnp.where(cols < C, lg, -jnp.inf)
    rowmax = jnp.max(lgm, axis=1, keepdims=True)
    idx = jnp.where(lgm == rowmax, cols, 2 * D)
    pred = jnp.min(idx, axis=1, keepdims=True)
    corr = (pred == lab_ref[...]).astype(jnp.float32)
    corr_ref[...] = corr

    @pl.when(i == 0)
    def _():
        acc_ref[...] = jnp.zeros_like(acc_ref)

    acc_ref[...] += jnp.sum(corr) * np.float32(1.0 / N)


def _tc_readout(raw_in, h, wg1, wg2, gb, wt, tb, labels2):
    full = lambda r, c: pl.BlockSpec((r, c), lambda i: (0, 0))
    return pl.pallas_call(
        _readout_body,
        grid=(NB,),
        in_specs=[
            pl.BlockSpec((BN, D), lambda i: (i, 0)),
            pl.BlockSpec((BN, D), lambda i: (i, 0)),
            full(D, D), full(D, D), full(1, D),
            full(D, D), full(1, D),
            pl.BlockSpec((BN, 1), lambda i: (i, 0)),
        ],
        out_specs=[
            pl.BlockSpec((BN, D), lambda i: (i, 0)),
            pl.BlockSpec((BN, 1), lambda i: (i, 0)),
            pl.BlockSpec((1, 1), lambda i: (0, 0)),
        ],
        out_shape=[
            jax.ShapeDtypeStruct((N, D), jnp.float32),
            jax.ShapeDtypeStruct((N, 1), jnp.float32),
            jax.ShapeDtypeStruct((1, 1), jnp.float32),
        ],
    )(raw_in, h, wg1, wg2, gb, wt, tb, labels2)


def kernel(vocab_ids, labels, edge_lists, emb, Wq, Wk, Wv, Wo, ln1_g, ln1_b,
           W1, b1, W2, b2, ln2_g, ln2_b, gate_W, gate_b, tr_W, tr_b):
    vocab_ids = vocab_ids.astype(jnp.int32)
    labels = labels.astype(jnp.int32)
    edge_lists = edge_lists.astype(jnp.int32)
    src_g = jnp.pad(edge_lists[0], (0, EPAD - E))
    dst_g = jnp.pad(edge_lists[1], (0, EPAD - E))
    dst_s = jnp.pad(edge_lists[1], (0, EPAD - E), constant_values=NACC - 1)
    dst_8 = jnp.right_shift(dst_s, 3)

    ids_pad = jnp.pad(vocab_ids, (0, NPAD - N))
    raw_in = _emb_gather(ids_pad, emb)[:N]

    sden = jnp.asarray(
        np.kron(np.concatenate([np.eye(H, dtype=np.float32),
                                np.zeros((H, H), np.float32)], axis=0),
                np.ones((1, DH), np.float32)))

    h = raw_in
    for i in range(L):
        wqkv = jnp.concatenate([Wq[i], Wk[i], Wv[i]], axis=1)
        q, kv = _tc_qkv(h, wqkv)
        nummsg, dens = _edge_attn(q, kv, dst_g, dst_s, dst_8, src_g)
        h = _tc_post(h, nummsg, dens, Wo[i],
                     ln1_g[i].reshape(1, D), ln1_b[i].reshape(1, D),
                     W1[i], b1[i].reshape(1, FF), W2[i],
                     b2[i].reshape(1, D),
                     ln2_g[i].reshape(1, D), ln2_b[i].reshape(1, D), sden)

    gwt = jnp.pad(gate_W.T, ((0, 0), (0, D - C)))
    wg1 = gwt[:D]
    wg2 = gwt[D:]
    gb = jnp.pad(gate_b, (0, D - C)).reshape(1, D)
    wt = jnp.pad(tr_W.T, ((0, 0), (0, D - C)))
    tb = jnp.pad(tr_b, (0, D - C)).reshape(1, D)

    logits_f, corr, accs = _tc_readout(raw_in, h, wg1, wg2, gb, wt, tb,
                                       labels.reshape(N, 1))
    logits = logits_f[:, :C]
    correct_preds = corr.reshape(N)
    accuracy = accs.reshape(())
    return (logits, accuracy, correct_preds, labels)


# speedup vs baseline: 14.6637x; 1.1579x over previous
import jax
import jax.numpy as jnp
import numpy as np
from jax import lax
from jax.experimental import pallas as pl
from jax.experimental.pallas import tpu as pltpu
from jax.experimental.pallas import tpu_sc as plsc

N = 10000
E = 320000
D = 128
H = 8
DH = D // H
L = 4
C = 104
FF = 512

NSC = 2
NTILE = 16
NW = NSC * NTILE
EPAD = 327680
EPW = EPAD // NW
CHUNK = 64
NCHUNK = EPW // CHUNK
TOTC = EPAD // CHUNK
NACC = 10240
RPT = NACC // NTILE
NDEN = NACC // 16
RPTD = NDEN // NTILE

NPAD = 10240
GPW = NPAD // NW


def _emb_body(ids_hbm, emb_hbm, out_hbm, idx_v, rows_v, sem):
    c = lax.axis_index("c")
    s = lax.axis_index("s")
    w = c * NTILE + s
    base = pl.multiple_of(w * GPW, 8)
    pltpu.sync_copy(ids_hbm.at[pl.ds(base, GPW)], idx_v)
    pltpu.async_copy(emb_hbm.at[idx_v], rows_v, sem).wait()
    pltpu.sync_copy(rows_v, out_hbm.at[pl.ds(base, GPW)])


def _emb_gather(ids_pad, emb):
    mesh = plsc.VectorSubcoreMesh(core_axis_name="c", subcore_axis_name="s")
    return pl.kernel(
        _emb_body,
        out_type=jax.ShapeDtypeStruct((NPAD, D), jnp.float32),
        mesh=mesh,
        scratch_types=[
            pltpu.VMEM((GPW,), jnp.int32),
            pltpu.VMEM((GPW, D), jnp.float32),
            pltpu.SemaphoreType.DMA,
        ],
    )(ids_pad, emb)


def _edge_body(q_hbm, kv_hbm, idx_hbm, out_hbm, den_hbm,
               idxr0, idxr1, qd, kvs, msg, denb, acc, accd, gsem, ssem):
    c = lax.axis_index("c")
    s = lax.axis_index("s")
    w = c * NTILE + s

    def zero_row(i, carry):
        for t in range(D // 16):
            msg[i, pl.ds(t * 16, 16)] = jnp.zeros((16,), jnp.float32)
        return carry

    lax.fori_loop(0, CHUNK, zero_row, 0)

    def zero_acc(t, carry):
        off = pl.multiple_of(s * RPT + t * CHUNK, 8)
        pltpu.sync_copy(msg.at[pl.ds(0, CHUNK)], acc.at[pl.ds(off, CHUNK)])
        return carry

    lax.fori_loop(0, RPT // CHUNK, zero_acc, 0)
    offd = pl.multiple_of(s * RPTD, 8)
    pltpu.sync_copy(msg.at[pl.ds(0, RPTD)], accd.at[pl.ds(offd, RPTD)])
    plsc.subcore_barrier()

    lane = lax.iota(jnp.int32, 16)
    dn = lax.GatherDimensionNumbers(offset_dims=(), collapsed_slice_dims=(0,),
                                    start_index_map=(0,))

    def _perm(v, idx):
        return lax.gather(v, idx[:, None], dn, (1,),
                          mode=lax.GatherScatterMode.PROMISE_IN_BOUNDS)

    lanef = lane.astype(jnp.float32)
    onehot = [jnp.maximum(1.0 - jnp.abs(lanef - float(hh)), 0.0)
              for hh in range(H)]
    shl8 = jnp.bitwise_and(lane + 8, 15)
    cbase = w * NCHUNK

    def _drain_scatters():
        pltpu.make_async_copy(q_hbm.at[pl.ds(0, CHUNK)], msg, ssem).wait()
        pltpu.make_async_copy(q_hbm.at[pl.ds(0, CHUNK)], denb, ssem).wait()

    def _run_chunk(j, idxr):
        pltpu.sync_copy(idx_hbm.at[cbase + j], idxr)
        cp1 = pltpu.async_copy(q_hbm.at[idxr.at[0]], qd, gsem)
        cp2 = pltpu.async_copy(kv_hbm.at[idxr.at[1]], kvs, gsem)

        @pl.when(j > 0)
        def _():
            _drain_scatters()

        cp1.wait()
        cp2.wait()

        def edge_body(e, carry2):
            den = jnp.zeros((16,), jnp.float32)
            for h in range(H):
                vq = qd[e, pl.ds(h * 16, 16)]
                vk = kvs[e, pl.ds(h * 16, 16)]
                vv = kvs[e, pl.ds(D + h * 16, 16)]
                p = vq * vk
                for k in (8, 4, 2, 1):
                    p = p + _perm(p, jnp.bitwise_xor(lane, k))
                ex = jnp.exp(p * 0.25)
                msg[e, pl.ds(h * 16, 16)] = ex * vv
                den = den + ex * onehot[h]
            dvi = idxr[4 + lax.shift_right_logical(e, 4), pl.ds(0, 16)]
            mf = _perm(dvi.astype(jnp.float32),
                       jnp.full((16,), jnp.bitwise_and(e, 15), jnp.int32))
            den_sh = _perm(den, shl8)
            for k in range(8):
                w0 = jnp.maximum(1.0 - jnp.abs(mf - float(2 * k)), 0.0)
                w1 = jnp.maximum(1.0 - jnp.abs(mf - float(2 * k + 1)), 0.0)
                denb[e, pl.ds(k * 16, 16)] = den * w0 + den_sh * w1
            return carry2

        lax.fori_loop(0, CHUNK, edge_body, 0)
        pltpu.async_copy(msg, acc.at[idxr.at[2]], ssem, add=True)
        pltpu.async_copy(denb, accd.at[idxr.at[3]], ssem, add=True)

    def chunk_body(j, carry):
        par = jnp.bitwise_and(j, 1)

        @pl.when(par == 0)
        def _():
            _run_chunk(j, idxr0)

        @pl.when(par == 1)
        def _():
            _run_chunk(j, idxr1)

        return carry

    lax.fori_loop(0, NCHUNK, chunk_body, 0)
    _drain_scatters()
    plsc.subcore_barrier()

    def copy_out(t, carry):
        off = pl.multiple_of(s * RPT + t * CHUNK, 8)
        pltpu.sync_copy(acc.at[pl.ds(off, CHUNK)],
                        out_hbm.at[c].at[pl.ds(off, CHUNK)])
        return carry

    lax.fori_loop(0, RPT // CHUNK, copy_out, 0)

    def unpack(t, carry):
        offp = pl.multiple_of(s * RPTD + t * 8, 8)
        pltpu.sync_copy(accd.at[pl.ds(offp, 8)], qd.at[pl.ds(0, 8)])
        for half in range(2):
            for r2 in range(4):
                pr = half * 4 + r2
                for m2 in range(16):
                    if m2 % 2 == 0:
                        v = qd[pr, pl.ds(8 * m2, 16)]
                    else:
                        v = _perm(qd[pr, pl.ds(8 * m2 - 8, 16)], shl8)
                    msg[r2 * 16 + m2, pl.ds(0, 16)] = v
            offn = pl.multiple_of(s * RPT + t * 128 + half * 64, 8)
            pltpu.sync_copy(msg, den_hbm.at[c].at[pl.ds(offn, 64)])
        return carry

    lax.fori_loop(0, RPTD // 8, unpack, 0)


def _build_idx(edge_lists):
    src_g = jnp.pad(edge_lists[0], (0, EPAD - E))
    dst_g = jnp.pad(edge_lists[1], (0, EPAD - E))
    dst_s = jnp.pad(edge_lists[1], (0, EPAD - E), constant_values=NACC - 1)
    dst16 = jnp.right_shift(dst_s, 4)
    mbits = jnp.bitwise_and(dst_s, 15)
    a = jnp.stack([dst_g, src_g, dst_s, dst16], 0).reshape(4, TOTC, CHUNK)
    a = a.transpose(1, 0, 2)
    mb = mbits.reshape(TOTC, CHUNK // 16, 16)
    mb = jnp.pad(mb, ((0, 0), (0, 0), (0, CHUNK - 16)))
    return jnp.concatenate([a, mb], axis=1)


def _edge_attn(q, kv, idx_all):
    mesh = plsc.VectorSubcoreMesh(core_axis_name="c", subcore_axis_name="s")
    return pl.kernel(
        _edge_body,
        out_type=(jax.ShapeDtypeStruct((NSC, NACC, D), jnp.float32),
                  jax.ShapeDtypeStruct((NSC, NACC, D), jnp.float32)),
        mesh=mesh,
        scratch_types=[
            pltpu.VMEM((8, CHUNK), jnp.int32),
            pltpu.VMEM((8, CHUNK), jnp.int32),
            pltpu.VMEM((CHUNK, D), jnp.float32),
            pltpu.VMEM((CHUNK, 2 * D), jnp.float32),
            pltpu.VMEM((CHUNK, D), jnp.float32),
            pltpu.VMEM((CHUNK, D), jnp.float32),
            pltpu.VMEM_SHARED((NACC, D), jnp.float32),
            pltpu.VMEM_SHARED((NDEN, D), jnp.float32),
            pltpu.SemaphoreType.DMA,
            pltpu.SemaphoreType.DMA,
        ],
    )(q, kv, idx_all)


BN = 1000
NB = N // BN


def _qkv_body(h_ref, w_ref, q_ref, kv_ref):
    res = jnp.dot(h_ref[...], w_ref[...], preferred_element_type=jnp.float32)
    q_ref[...] = res[:, :D]
    kv_ref[...] = res[:, D:]


def _tc_qkv(h, wqkv):
    return pl.pallas_call(
        _qkv_body,
        grid=(NB,),
        in_specs=[
            pl.BlockSpec((BN, D), lambda i: (i, 0)),
            pl.BlockSpec((D, 3 * D), lambda i: (0, 0)),
        ],
        out_specs=[
            pl.BlockSpec((BN, D), lambda i: (i, 0)),
            pl.BlockSpec((BN, 2 * D), lambda i: (i, 0)),
        ],
        out_shape=[
            jax.ShapeDtypeStruct((N, D), jnp.float32),
            jax.ShapeDtypeStruct((N, 2 * D), jnp.float32),
        ],
    )(h, wqkv)


def _ln(x, g, b):
    mu = jnp.mean(x, axis=-1, keepdims=True)
    r = x - mu
    var = jnp.mean(r * r, axis=-1, keepdims=True)
    return r / jnp.sqrt(var + 1e-5) * g + b


def _post_body(h_ref, nd0_ref, nd1_ref, de0_ref, de1_ref, wo_ref,
               ln1g_ref, ln1b_ref,
               w1_ref, b1_ref, w2_ref, b2_ref, ln2g_ref, ln2b_ref,
               sden_ref, out_ref):
    num = (nd0_ref[...] + nd1_ref[...])[0]
    den16 = (de0_ref[...] + de1_ref[...])[0][:, :16]
    denb = jnp.dot(den16, sden_ref[...], preferred_element_type=jnp.float32)
    agg = num / (denb + 1e-9)
    attn = jnp.dot(agg, wo_ref[...], preferred_element_type=jnp.float32)
    x = _ln(h_ref[...] + attn, ln1g_ref[...], ln1b_ref[...])
    ffp = jnp.maximum(
        jnp.dot(x, w1_ref[...], preferred_element_type=jnp.float32)
        + b1_ref[...], 0.0)
    ff = jnp.dot(ffp, w2_ref[...], preferred_element_type=jnp.float32) \
        + b2_ref[...]
    out_ref[...] = _ln(x + ff, ln2g_ref[...], ln2b_ref[...])


def _tc_post(h, nummsg, dens, wo, ln1g, ln1b, w1, b1, w2, b2, ln2g, ln2b,
             sden):
    full = lambda r, c: pl.BlockSpec((r, c), lambda i: (0, 0))
    return pl.pallas_call(
        _post_body,
        grid=(NB,),
        in_specs=[
            pl.BlockSpec((BN, D), lambda i: (i, 0)),
            pl.BlockSpec((1, BN, D), lambda i: (0, i, 0)),
            pl.BlockSpec((1, BN, D), lambda i: (1, i, 0)),
            pl.BlockSpec((1, BN, D), lambda i: (0, i, 0)),
            pl.BlockSpec((1, BN, D), lambda i: (1, i, 0)),
            full(D, D),
            full(1, D), full(1, D),
            full(D, FF), full(1, FF),
            full(FF, D), full(1, D),
            full(1, D), full(1, D),
            full(16, D),
        ],
        out_specs=pl.BlockSpec((BN, D), lambda i: (i, 0)),
        out_shape=jax.ShapeDtypeStruct((N, D), jnp.float32),
    )(h, nummsg, nummsg, dens, dens, wo, ln1g, ln1b, w1, b1, w2, b2,
      ln2g, ln2b, sden)


def _readout_body(ri_ref, h_ref, wg1_ref, wg2_ref, gb_ref, wt_ref, tb_ref,
                  lab_ref, logit_ref, corr_ref, acc_ref):
    i = pl.program_id(0)
    ri = ri_ref[...]
    hh = h_ref[...]
    g = jax.nn.sigmoid(
        jnp.dot(ri, wg1_ref[...], preferred_element_type=jnp.float32)
        + jnp.dot(hh, wg2_ref[...], preferred_element_type=jnp.float32)
        + gb_ref[...])
    t = jnp.dot(hh, wt_ref[...], preferred_element_type=jnp.float32) \
        + tb_ref[...]
    lg = g * t
    logit_ref[...] = lg
    cols = lax.broadcasted_iota(jnp.int32, lg.shape, 1)
    lgm = jnp.where(cols < C, lg, -jnp.inf)
    rowmax = jnp.max(lgm, axis=1, keepdims=True)
    idx = jnp.where(lgm == rowmax, cols, 2 * D)
    pred = jnp.min(idx, axis=1, keepdims=True)
    corr = (pred == lab_ref[...]).astype(jnp.float32)
    corr_ref[...] = corr

    @pl.when(i == 0)
    def _():
        acc_ref[...] = jnp.zeros_like(acc_ref)

    acc_ref[...] += jnp.sum(corr) * np.float32(1.0 / N)


def _tc_readout(raw_in, h, wg1, wg2, gb, wt, tb, labels2):
    full = lambda r, c: pl.BlockSpec((r, c), lambda i: (0, 0))
    return pl.pallas_call(
        _readout_body,
        grid=(NB,),
        in_specs=[
            pl.BlockSpec((BN, D), lambda i: (i, 0)),
            pl.BlockSpec((BN, D), lambda i: (i, 0)),
            full(D, D), full(D, D), full(1, D),
            full(D, D), full(1, D),
            pl.BlockSpec((BN, 1), lambda i: (i, 0)),
        ],
        out_specs=[
            pl.BlockSpec((BN, D), lambda i: (i, 0)),
            pl.BlockSpec((BN, 1), lambda i: (i, 0)),
            pl.BlockSpec((1, 1), lambda i: (0, 0)),
        ],
        out_shape=[
            jax.ShapeDtypeStruct((N, D), jnp.float32),
            jax.ShapeDtypeStruct((N, 1), jnp.float32),
            jax.ShapeDtypeStruct((1, 1), jnp.float32),
        ],
    )(raw_in, h, wg1, wg2, gb, wt, tb, labels2)


def kernel(vocab_ids, labels, edge_lists, emb, Wq, Wk, Wv, Wo, ln1_g, ln1_b,
           W1, b1, W2, b2, ln2_g, ln2_b, gate_W, gate_b, tr_W, tr_b):
    vocab_ids = vocab_ids.astype(jnp.int32)
    labels = labels.astype(jnp.int32)
    edge_lists = edge_lists.astype(jnp.int32)
    idx_all = _build_idx(edge_lists)

    ids_pad = jnp.pad(vocab_ids, (0, NPAD - N))
    raw_in = _emb_gather(ids_pad, emb)[:N]

    sden = jnp.asarray(
        np.kron(np.concatenate([np.eye(H, dtype=np.float32),
                                np.zeros((H, H), np.float32)], axis=0),
                np.ones((1, DH), np.float32)))

    h = raw_in
    for i in range(L):
        wqkv = jnp.concatenate([Wq[i], Wk[i], Wv[i]], axis=1)
        q, kv = _tc_qkv(h, wqkv)
        nummsg, dens = _edge_attn(q, kv, idx_all)
        h = _tc_post(h, nummsg, dens, Wo[i],
                     ln1_g[i].reshape(1, D), ln1_b[i].reshape(1, D),
                     W1[i], b1[i].reshape(1, FF), W2[i],
                     b2[i].reshape(1, D),
                     ln2_g[i].reshape(1, D), ln2_b[i].reshape(1, D), sden)

    gwt = jnp.pad(gate_W.T, ((0, 0), (0, D - C)))
    wg1 = gwt[:D]
    wg2 = gwt[D:]
    gb = jnp.pad(gate_b, (0, D - C)).reshape(1, D)
    wt = jnp.pad(tr_W.T, ((0, 0), (0, D - C)))
    tb = jnp.pad(tr_b, (0, D - C)).reshape(1, D)

    logits_f, corr, accs = _tc_readout(raw_in, h, wg1, wg2, gb, wt, tb,
                                       labels.reshape(N, 1))
    logits = logits_f[:, :C]
    correct_preds = corr.reshape(N)
    accuracy = accs.reshape(())
    return (logits, accuracy, correct_preds, labels)


# speedup vs baseline: 24.1293x; 1.6455x over previous
import jax
import jax.numpy as jnp
import numpy as np
from jax import lax
from jax.experimental import pallas as pl
from jax.experimental.pallas import tpu as pltpu
from jax.experimental.pallas import tpu_sc as plsc

N = 10000
E = 320000
D = 128
H = 8
DH = D // H
L = 4
C = 104
FF = 512

NSC = 2
NTILE = 16
NW = NSC * NTILE
EPAD = 327680
EPW = EPAD // NW
CHUNK = 64
NCHUNK = EPW // CHUNK
TOTC = EPAD // CHUNK
NACC = 10240
RPT = NACC // NTILE
NDEN = NACC // 16
RPTD = NDEN // NTILE

NPAD = 10240
GPW = NPAD // NW


def _emb_body(ids_hbm, emb_hbm, out_hbm, idx_v, rows_v, sem):
    c = lax.axis_index("c")
    s = lax.axis_index("s")
    w = c * NTILE + s
    base = pl.multiple_of(w * GPW, 8)
    pltpu.sync_copy(ids_hbm.at[pl.ds(base, GPW)], idx_v)
    pltpu.async_copy(emb_hbm.at[idx_v], rows_v, sem).wait()
    pltpu.sync_copy(rows_v, out_hbm.at[pl.ds(base, GPW)])


def _emb_gather(ids_pad, emb):
    mesh = plsc.VectorSubcoreMesh(core_axis_name="c", subcore_axis_name="s")
    return pl.kernel(
        _emb_body,
        out_type=jax.ShapeDtypeStruct((NPAD, D), jnp.float32),
        mesh=mesh,
        scratch_types=[
            pltpu.VMEM((GPW,), jnp.int32),
            pltpu.VMEM((GPW, D), jnp.float32),
            pltpu.SemaphoreType.DMA,
        ],
    )(ids_pad, emb)


def _edge_body(q_hbm, kv_hbm, idx_hbm, out_hbm, den_hbm,
               idxr0, idxr1, qd, kvs, msg, denb, acc, accd, gsem, ssem):
    c = lax.axis_index("c")
    s = lax.axis_index("s")
    w = c * NTILE + s

    def zero_row(i, carry):
        for t in range(D // 16):
            msg[i, pl.ds(t * 16, 16)] = jnp.zeros((16,), jnp.float32)
        return carry

    lax.fori_loop(0, CHUNK, zero_row, 0)

    def zero_acc(t, carry):
        off = pl.multiple_of(s * RPT + t * CHUNK, 8)
        pltpu.sync_copy(msg.at[pl.ds(0, CHUNK)], acc.at[pl.ds(off, CHUNK)])
        return carry

    lax.fori_loop(0, RPT // CHUNK, zero_acc, 0)
    offd = pl.multiple_of(s * RPTD, 8)
    pltpu.sync_copy(msg.at[pl.ds(0, RPTD)], accd.at[pl.ds(offd, RPTD)])
    plsc.subcore_barrier()

    lane = lax.iota(jnp.int32, 16)
    dn = lax.GatherDimensionNumbers(offset_dims=(), collapsed_slice_dims=(0,),
                                    start_index_map=(0,))

    def _perm(v, idx):
        return lax.gather(v, idx[:, None], dn, (1,),
                          mode=lax.GatherScatterMode.PROMISE_IN_BOUNDS)

    lanef = lane.astype(jnp.float32)
    onehot = [jnp.maximum(1.0 - jnp.abs(lanef - float(hh)), 0.0)
              for hh in range(H)]
    shl8 = jnp.bitwise_and(lane + 8, 15)
    cbase = w * NCHUNK

    def _drain_scatters():
        pltpu.make_async_copy(q_hbm.at[pl.ds(0, CHUNK)], msg, ssem).wait()
        pltpu.make_async_copy(q_hbm.at[pl.ds(0, CHUNK)], denb, ssem).wait()

    def _run_chunk(j, idxr):
        pltpu.sync_copy(idx_hbm.at[cbase + j], idxr)
        cp1 = pltpu.async_copy(q_hbm.at[idxr.at[0]], qd, gsem)
        cp2 = pltpu.async_copy(kv_hbm.at[idxr.at[1]], kvs, gsem)

        @pl.when(j > 0)
        def _():
            _drain_scatters()

        cp1.wait()
        cp2.wait()

        @plsc.parallel_loop(0, CHUNK, 1, unroll=2)
        def edge_body(e):
            den = jnp.zeros((16,), jnp.float32)
            for h in range(H):
                vq = qd[e, pl.ds(h * 16, 16)]
                vk = kvs[e, pl.ds(h * 16, 16)]
                vv = kvs[e, pl.ds(D + h * 16, 16)]
                p = vq * vk
                for k in (8, 4, 2, 1):
                    p = p + _perm(p, jnp.bitwise_xor(lane, k))
                ex = jnp.exp(p * 0.25)
                msg[e, pl.ds(h * 16, 16)] = ex * vv
                den = den + ex * onehot[h]
            dvi = idxr[4 + lax.shift_right_logical(e, 4), pl.ds(0, 16)]
            mf = _perm(dvi.astype(jnp.float32),
                       jnp.full((16,), jnp.bitwise_and(e, 15), jnp.int32))
            den_sh = _perm(den, shl8)
            for k in range(8):
                w0 = jnp.maximum(1.0 - jnp.abs(mf - float(2 * k)), 0.0)
                w1 = jnp.maximum(1.0 - jnp.abs(mf - float(2 * k + 1)), 0.0)
                denb[e, pl.ds(k * 16, 16)] = den * w0 + den_sh * w1

        pltpu.async_copy(msg, acc.at[idxr.at[2]], ssem, add=True)
        pltpu.async_copy(denb, accd.at[idxr.at[3]], ssem, add=True)

    def chunk_body(j, carry):
        par = jnp.bitwise_and(j, 1)

        @pl.when(par == 0)
        def _():
            _run_chunk(j, idxr0)

        @pl.when(par == 1)
        def _():
            _run_chunk(j, idxr1)

        return carry

    lax.fori_loop(0, NCHUNK, chunk_body, 0)
    _drain_scatters()
    plsc.subcore_barrier()

    def copy_out(t, carry):
        off = pl.multiple_of(s * RPT + t * CHUNK, 8)
        pltpu.sync_copy(acc.at[pl.ds(off, CHUNK)],
                        out_hbm.at[c].at[pl.ds(off, CHUNK)])
        return carry

    lax.fori_loop(0, RPT // CHUNK, copy_out, 0)

    def unpack(t, carry):
        offp = pl.multiple_of(s * RPTD + t * 8, 8)
        pltpu.sync_copy(accd.at[pl.ds(offp, 8)], qd.at[pl.ds(0, 8)])
        for half in range(2):
            for r2 in range(4):
                pr = half * 4 + r2
                for m2 in range(16):
                    if m2 % 2 == 0:
                        v = qd[pr, pl.ds(8 * m2, 16)]
                    else:
                        v = _perm(qd[pr, pl.ds(8 * m2 - 8, 16)], shl8)
                    msg[r2 * 16 + m2, pl.ds(0, 16)] = v
            offn = pl.multiple_of(s * RPT + t * 128 + half * 64, 8)
            pltpu.sync_copy(msg, den_hbm.at[c].at[pl.ds(offn, 64)])
        return carry

    lax.fori_loop(0, RPTD // 8, unpack, 0)


def _build_idx(edge_lists):
    src_g = jnp.pad(edge_lists[0], (0, EPAD - E))
    dst_g = jnp.pad(edge_lists[1], (0, EPAD - E))
    dst_s = jnp.pad(edge_lists[1], (0, EPAD - E), constant_values=NACC - 1)
    dst16 = jnp.right_shift(dst_s, 4)
    mbits = jnp.bitwise_and(dst_s, 15)
    a = jnp.stack([dst_g, src_g, dst_s, dst16], 0).reshape(4, TOTC, CHUNK)
    a = a.transpose(1, 0, 2)
    mb = mbits.reshape(TOTC, CHUNK // 16, 16)
    mb = jnp.pad(mb, ((0, 0), (0, 0), (0, CHUNK - 16)))
    return jnp.concatenate([a, mb], axis=1)


def _edge_attn(q, kv, idx_all):
    mesh = plsc.VectorSubcoreMesh(core_axis_name="c", subcore_axis_name="s")
    return pl.kernel(
        _edge_body,
        out_type=(jax.ShapeDtypeStruct((NSC, NACC, D), jnp.float32),
                  jax.ShapeDtypeStruct((NSC, NACC, D), jnp.float32)),
        mesh=mesh,
        scratch_types=[
            pltpu.VMEM((8, CHUNK), jnp.int32),
            pltpu.VMEM((8, CHUNK), jnp.int32),
            pltpu.VMEM((CHUNK, D), jnp.float32),
            pltpu.VMEM((CHUNK, 2 * D), jnp.float32),
            pltpu.VMEM((CHUNK, D), jnp.float32),
            pltpu.VMEM((CHUNK, D), jnp.float32),
            pltpu.VMEM_SHARED((NACC, D), jnp.float32),
            pltpu.VMEM_SHARED((NDEN, D), jnp.float32),
            pltpu.SemaphoreType.DMA,
            pltpu.SemaphoreType.DMA,
        ],
    )(q, kv, idx_all)


BN = 1000
NB = N // BN


def _qkv_body(h_ref, w_ref, q_ref, kv_ref):
    res = jnp.dot(h_ref[...], w_ref[...], preferred_element_type=jnp.float32)
    q_ref[...] = res[:, :D]
    kv_ref[...] = res[:, D:]


def _tc_qkv(h, wqkv):
    return pl.pallas_call(
        _qkv_body,
        grid=(NB,),
        in_specs=[
            pl.BlockSpec((BN, D), lambda i: (i, 0)),
            pl.BlockSpec((D, 3 * D), lambda i: (0, 0)),
        ],
        out_specs=[
            pl.BlockSpec((BN, D), lambda i: (i, 0)),
            pl.BlockSpec((BN, 2 * D), lambda i: (i, 0)),
        ],
        out_shape=[
            jax.ShapeDtypeStruct((N, D), jnp.float32),
            jax.ShapeDtypeStruct((N, 2 * D), jnp.float32),
        ],
    )(h, wqkv)


def _ln(x, g, b):
    mu = jnp.mean(x, axis=-1, keepdims=True)
    r = x - mu
    var = jnp.mean(r * r, axis=-1, keepdims=True)
    return r / jnp.sqrt(var + 1e-5) * g + b


def _post_body(h_ref, nd0_ref, nd1_ref, de0_ref, de1_ref, wo_ref,
               ln1g_ref, ln1b_ref,
               w1_ref, b1_ref, w2_ref, b2_ref, ln2g_ref, ln2b_ref,
               sden_ref, out_ref):
    num = (nd0_ref[...] + nd1_ref[...])[0]
    den16 = (de0_ref[...] + de1_ref[...])[0][:, :16]
    denb = jnp.dot(den16, sden_ref[...], preferred_element_type=jnp.float32)
    agg = num / (denb + 1e-9)
    attn = jnp.dot(agg, wo_ref[...], preferred_element_type=jnp.float32)
    x = _ln(h_ref[...] + attn, ln1g_ref[...], ln1b_ref[...])
    ffp = jnp.maximum(
        jnp.dot(x, w1_ref[...], preferred_element_type=jnp.float32)
        + b1_ref[...], 0.0)
    ff = jnp.dot(ffp, w2_ref[...], preferred_element_type=jnp.float32) \
        + b2_ref[...]
    out_ref[...] = _ln(x + ff, ln2g_ref[...], ln2b_ref[...])


def _tc_post(h, nummsg, dens, wo, ln1g, ln1b, w1, b1, w2, b2, ln2g, ln2b,
             sden):
    full = lambda r, c: pl.BlockSpec((r, c), lambda i: (0, 0))
    return pl.pallas_call(
        _post_body,
        grid=(NB,),
        in_specs=[
            pl.BlockSpec((BN, D), lambda i: (i, 0)),
            pl.BlockSpec((1, BN, D), lambda i: (0, i, 0)),
            pl.BlockSpec((1, BN, D), lambda i: (1, i, 0)),
            pl.BlockSpec((1, BN, D), lambda i: (0, i, 0)),
            pl.BlockSpec((1, BN, D), lambda i: (1, i, 0)),
            full(D, D),
            full(1, D), full(1, D),
            full(D, FF), full(1, FF),
            full(FF, D), full(1, D),
            full(1, D), full(1, D),
            full(16, D),
        ],
        out_specs=pl.BlockSpec((BN, D), lambda i: (i, 0)),
        out_shape=jax.ShapeDtypeStruct((N, D), jnp.float32),
    )(h, nummsg, nummsg, dens, dens, wo, ln1g, ln1b, w1, b1, w2, b2,
      ln2g, ln2b, sden)


def _readout_body(ri_ref, h_ref, wg1_ref, wg2_ref, gb_ref, wt_ref, tb_ref,
                  lab_ref, logit_ref, corr_ref, acc_ref):
    i = pl.program_id(0)
    ri = ri_ref[...]
    hh = h_ref[...]
    g = jax.nn.sigmoid(
        jnp.dot(ri, wg1_ref[...], preferred_element_type=jnp.float32)
        + jnp.dot(hh, wg2_ref[...], preferred_element_type=jnp.float32)
        + gb_ref[...])
    t = jnp.dot(hh, wt_ref[...], preferred_element_type=jnp.float32) \
        + tb_ref[...]
    lg = g * t
    logit_ref[...] = lg
    cols = lax.broadcasted_iota(jnp.int32, lg.shape, 1)
    lgm = jnp.where(cols < C, lg, -jnp.inf)
    rowmax = jnp.max(lgm, axis=1, keepdims=True)
    idx = jnp.where(lgm == rowmax, cols, 2 * D)
    pred = jnp.min(idx, axis=1, keepdims=True)
    corr = (pred == lab_ref[...]).astype(jnp.float32)
    corr_ref[...] = corr

    @pl.when(i == 0)
    def _():
        acc_ref[...] = jnp.zeros_like(acc_ref)

    acc_ref[...] += jnp.sum(corr) * np.float32(1.0 / N)


def _tc_readout(raw_in, h, wg1, wg2, gb, wt, tb, labels2):
    full = lambda r, c: pl.BlockSpec((r, c), lambda i: (0, 0))
    return pl.pallas_call(
        _readout_body,
        grid=(NB,),
        in_specs=[
            pl.BlockSpec((BN, D), lambda i: (i, 0)),
            pl.BlockSpec((BN, D), lambda i: (i, 0)),
            full(D, D), full(D, D), full(1, D),
            full(D, D), full(1, D),
            pl.BlockSpec((BN, 1), lambda i: (i, 0)),
        ],
        out_specs=[
            pl.BlockSpec((BN, D), lambda i: (i, 0)),
            pl.BlockSpec((BN, 1), lambda i: (i, 0)),
            pl.BlockSpec((1, 1), lambda i: (0, 0)),
        ],
        out_shape=[
            jax.ShapeDtypeStruct((N, D), jnp.float32),
            jax.ShapeDtypeStruct((N, 1), jnp.float32),
            jax.ShapeDtypeStruct((1, 1), jnp.float32),
        ],
    )(raw_in, h, wg1, wg2, gb, wt, tb, labels2)


def kernel(vocab_ids, labels, edge_lists, emb, Wq, Wk, Wv, Wo, ln1_g, ln1_b,
           W1, b1, W2, b2, ln2_g, ln2_b, gate_W, gate_b, tr_W, tr_b):
    vocab_ids = vocab_ids.astype(jnp.int32)
    labels = labels.astype(jnp.int32)
    edge_lists = edge_lists.astype(jnp.int32)
    idx_all = _build_idx(edge_lists)

    ids_pad = jnp.pad(vocab_ids, (0, NPAD - N))
    raw_in = _emb_gather(ids_pad, emb)[:N]

    sden = jnp.asarray(
        np.kron(np.concatenate([np.eye(H, dtype=np.float32),
                                np.zeros((H, H), np.float32)], axis=0),
                np.ones((1, DH), np.float32)))

    h = raw_in
    for i in range(L):
        wqkv = jnp.concatenate([Wq[i], Wk[i], Wv[i]], axis=1)
        q, kv = _tc_qkv(h, wqkv)
        nummsg, dens = _edge_attn(q, kv, idx_all)
        h = _tc_post(h, nummsg, dens, Wo[i],
                     ln1_g[i].reshape(1, D), ln1_b[i].reshape(1, D),
                     W1[i], b1[i].reshape(1, FF), W2[i],
                     b2[i].reshape(1, D),
                     ln2_g[i].reshape(1, D), ln2_b[i].reshape(1, D), sden)

    gwt = jnp.pad(gate_W.T, ((0, 0), (0, D - C)))
    wg1 = gwt[:D]
    wg2 = gwt[D:]
    gb = jnp.pad(gate_b, (0, D - C)).reshape(1, D)
    wt = jnp.pad(tr_W.T, ((0, 0), (0, D - C)))
    tb = jnp.pad(tr_b, (0, D - C)).reshape(1, D)

    logits_f, corr, accs = _tc_readout(raw_in, h, wg1, wg2, gb, wt, tb,
                                       labels.reshape(N, 1))
    logits = logits_f[:, :C]
    correct_preds = corr.reshape(N)
    accuracy = accs.reshape(())
    return (logits, accuracy, correct_preds, labels)
